# Initial kernel scaffold; baseline (speedup 1.0000x reference)
import jax
import jax.numpy as jnp
from jax.experimental import pallas as pl


def kernel(node_features, edge_index, gnn_W0, gnn_b0, gnn_W1, gnn_b1, gnn_W2, gnn_b2, gnn_W3, gnn_b3, actor_W0, actor_b0, actor_W1, actor_b1, actor_W2, actor_b2, critic_W0, critic_b0, critic_W1, critic_b1, critic_W2, critic_b2):
    raise NotImplementedError("write your pallas kernel here")


# speedup vs baseline: 2.7761x; 2.7761x over previous
import functools

import jax
import jax.numpy as jnp
from jax import lax
from jax.experimental import pallas as pl
from jax.experimental.pallas import tpu as pltpu
from jax.experimental.pallas import tpu_sc as plsc

N = 10000
E = 320000
NCORE = 2
NSUB = 16
EDGES_PER_SUB = E // NSUB
EB = 80
NSUB_IO = 10
ROWS_IO = N // NSUB_IO

_f32 = jnp.float32


def _sc_mesh():
    return plsc.VectorSubcoreMesh(
        core_axis_name="c", subcore_axis_name="s",
        num_cores=NCORE, num_subcores=NSUB)


def _deg_body(dst_hbm, ones_hbm, zeros_hbm, out_hbm, dst_v, ones_v, acc, sem):
    c = lax.axis_index("c")
    s = lax.axis_index("s")
    pltpu.sync_copy(ones_hbm, ones_v)

    @pl.when(s < NSUB_IO)
    def _():
        pltpu.sync_copy(zeros_hbm, acc.at[pl.ds(s * ROWS_IO, ROWS_IO)])

    plsc.subcore_barrier()

    half = E // NCORE
    per_sub = half // NSUB

    def step(k, carry):
        e0 = c * half + s * per_sub + k * EB
        pltpu.sync_copy(dst_hbm.at[pl.ds(e0, EB)], dst_v)
        pltpu.sync_copy(ones_v, acc.at[dst_v], add=True)
        return carry

    lax.fori_loop(0, per_sub // EB, step, 0)
    plsc.subcore_barrier()

    @pl.when(s < NSUB_IO)
    def _():
        r0 = s * ROWS_IO
        pltpu.sync_copy(acc.at[pl.ds(r0, ROWS_IO)],
                        out_hbm.at[pl.ds(c * N + r0, ROWS_IO)])


def _deg_kernel(dst, ones128, zeros128):
    fn = pl.kernel(
        _deg_body,
        out_type=jax.ShapeDtypeStruct((NCORE * N, 128), _f32),
        mesh=_sc_mesh(),
        scratch_types=[
            pltpu.VMEM((EB,), jnp.int32),
            pltpu.VMEM((EB, 128), _f32),
            pltpu.VMEM_SHARED((N, 128), _f32),
            pltpu.SemaphoreType.DMA,
        ],
    )
    return fn(dst, ones128, zeros128)


def _segsum_body(srcs_hbm, dst_hbm, x_hbm, zeros_hbm, out_hbm,
                 idx_v, dst_v, rows_v, acc, sem, *, ncn):
    c = lax.axis_index("c")
    s = lax.axis_index("s")
    r0 = s * ROWS_IO

    for ci in range(ncn // NCORE):
        chunk = 2 * ci + c

        @pl.when(s < NSUB_IO)
        def _():
            pltpu.sync_copy(zeros_hbm, acc.at[pl.ds(r0, ROWS_IO)])

        plsc.subcore_barrier()

        def step(k, carry):
            e0 = s * EDGES_PER_SUB + k * EB
            pltpu.sync_copy(srcs_hbm.at[pl.ds(chunk * E + e0, EB)], idx_v)
            pltpu.async_copy(x_hbm.at[idx_v], rows_v, sem).wait()
            pltpu.sync_copy(dst_hbm.at[pl.ds(e0, EB)], dst_v)
            pltpu.sync_copy(rows_v, acc.at[dst_v], add=True)
            return carry

        lax.fori_loop(0, EDGES_PER_SUB // EB, step, 0)
        plsc.subcore_barrier()

        @pl.when(s < NSUB_IO)
        def _():
            pltpu.sync_copy(acc.at[pl.ds(r0, ROWS_IO)],
                            out_hbm.at[pl.ds(chunk * N + r0, ROWS_IO)])

        plsc.subcore_barrier()


def _segsum_split_body(src_hbm, dst_hbm, x_hbm, zeros_hbm, out_hbm,
                       idx_v, dst_v, rows_v, acc, sem):
    c = lax.axis_index("c")
    s = lax.axis_index("s")
    r0 = s * ROWS_IO

    @pl.when(s < NSUB_IO)
    def _():
        pltpu.sync_copy(zeros_hbm, acc.at[pl.ds(r0, ROWS_IO)])

    plsc.subcore_barrier()

    half = E // NCORE
    per_sub = half // NSUB

    def step(k, carry):
        e0 = c * half + s * per_sub + k * EB
        pltpu.sync_copy(src_hbm.at[pl.ds(e0, EB)], idx_v)
        pltpu.async_copy(x_hbm.at[idx_v], rows_v, sem).wait()
        pltpu.sync_copy(dst_hbm.at[pl.ds(e0, EB)], dst_v)
        pltpu.sync_copy(rows_v, acc.at[dst_v], add=True)
        return carry

    lax.fori_loop(0, per_sub // EB, step, 0)
    plsc.subcore_barrier()

    @pl.when(s < NSUB_IO)
    def _():
        pltpu.sync_copy(acc.at[pl.ds(r0, ROWS_IO)],
                        out_hbm.at[pl.ds(c * N + r0, ROWS_IO)])


def _segsum_split(src, dst, x, zeros_fc):
    fn = pl.kernel(
        _segsum_split_body,
        out_type=jax.ShapeDtypeStruct((NCORE * N, 128), _f32),
        mesh=_sc_mesh(),
        scratch_types=[
            pltpu.VMEM((EB,), jnp.int32),
            pltpu.VMEM((EB,), jnp.int32),
            pltpu.VMEM((EB, 128), _f32),
            pltpu.VMEM_SHARED((N, 128), _f32),
            pltpu.SemaphoreType.DMA,
        ],
    )
    return fn(src, dst, x, zeros_fc)


def _segsum(srcs, dst, x_st, zeros_fc, ncn, fc):
    fn = pl.kernel(
        functools.partial(_segsum_body, ncn=ncn),
        out_type=jax.ShapeDtypeStruct((ncn * N, fc), _f32),
        mesh=_sc_mesh(),
        scratch_types=[
            pltpu.VMEM((EB,), jnp.int32),
            pltpu.VMEM((EB,), jnp.int32),
            pltpu.VMEM((EB, fc), _f32),
            pltpu.VMEM_SHARED((N, fc), _f32),
            pltpu.SemaphoreType.DMA,
        ],
    )
    return fn(srcs, dst, x_st, zeros_fc)


RB = 400
NRB = N // RB


def _layer_body(x_ref, s_ref, d0_ref, d1_ref, w_ref, b_ref, o_ref, *, ncin):
    ji = pl.program_id(2)
    deg = jnp.maximum(d0_ref[:, :1] + d1_ref[:, :1], 1.0)
    m = x_ref[...] + s_ref[...] / deg
    prod = jnp.dot(m, w_ref[...], preferred_element_type=_f32)

    @pl.when(ji == 0)
    def _():
        o_ref[...] = prod

    @pl.when(ji > 0)
    def _():
        o_ref[...] += prod

    @pl.when(ji == ncin - 1)
    def _():
        o_ref[...] = jnp.maximum(o_ref[...] + b_ref[...], 0.0)


def _layer0_body(x_ref, s0_ref, s1_ref, d0_ref, d1_ref, w_ref, b_ref, o_ref):
    deg = jnp.maximum(d0_ref[:, :1] + d1_ref[:, :1], 1.0)
    m = x_ref[...] + (s0_ref[...] + s1_ref[...]) / deg
    prod = jnp.dot(m, w_ref[...], preferred_element_type=_f32)
    o_ref[...] = jnp.maximum(prod + b_ref[...], 0.0)


def _tc_layer0(x, s_parts, degp, W, b, ncout):
    return pl.pallas_call(
        _layer0_body,
        grid=(NRB, ncout),
        in_specs=[
            pl.BlockSpec((RB, 128), lambda i, jo: (i, 0)),
            pl.BlockSpec((RB, 128), lambda i, jo: (i, 0)),
            pl.BlockSpec((RB, 128), lambda i, jo: (NRB + i, 0)),
            pl.BlockSpec((RB, 128), lambda i, jo: (i, 0)),
            pl.BlockSpec((RB, 128), lambda i, jo: (NRB + i, 0)),
            pl.BlockSpec((128, 128), lambda i, jo: (0, jo)),
            pl.BlockSpec((1, 128), lambda i, jo: (0, jo)),
        ],
        out_specs=pl.BlockSpec((RB, 128), lambda i, jo: (jo * NRB + i, 0)),
        out_shape=jax.ShapeDtypeStruct((ncout * N, 128), _f32),
    )(x, s_parts, s_parts, degp, degp, W, b.reshape(1, -1))


def _tc_layer(x_st, s_st, degp, W, b, ncin, fc_in, ncout):
    fc_out = 128
    return pl.pallas_call(
        functools.partial(_layer_body, ncin=ncin),
        grid=(NRB, ncout, ncin),
        in_specs=[
            pl.BlockSpec((RB, fc_in), lambda i, jo, ji: (ji * NRB + i, 0)),
            pl.BlockSpec((RB, fc_in), lambda i, jo, ji: (ji * NRB + i, 0)),
            pl.BlockSpec((RB, 128), lambda i, jo, ji: (i, 0)),
            pl.BlockSpec((RB, 128), lambda i, jo, ji: (NRB + i, 0)),
            pl.BlockSpec((fc_in, fc_out), lambda i, jo, ji: (ji, jo)),
            pl.BlockSpec((1, fc_out), lambda i, jo, ji: (0, jo)),
        ],
        out_specs=pl.BlockSpec((RB, fc_out), lambda i, jo, ji: (jo * NRB + i, 0)),
        out_shape=jax.ShapeDtypeStruct((ncout * N, fc_out), _f32),
    )(x_st, s_st, degp, degp, W, b.reshape(1, -1))


def _head_body(xa_ref, xb_ref, aw0, ab0, aw1, ab1, aw2, ab2,
               cw0, cb0, cw1, cb1, cw2, cb2, logits_ref, val_ref, g_acc):
    i = pl.program_id(0)

    @pl.when(i == 0)
    def _():
        g_acc[...] = jnp.zeros_like(g_acc)

    g_acc[0:1, 0:128] += jnp.sum(xa_ref[...], axis=0, keepdims=True)
    g_acc[0:1, 128:256] += jnp.sum(xb_ref[...], axis=0, keepdims=True)

    @pl.when(i == NRB - 1)
    def _():
        g = g_acc[...] * (1.0 / N)
        h = jnp.maximum(jnp.dot(g, aw0[...], preferred_element_type=_f32)
                        + ab0[...], 0.0)
        h = jnp.maximum(jnp.dot(h, aw1[...], preferred_element_type=_f32)
                        + ab1[...], 0.0)
        logits_ref[...] = jnp.dot(h, aw2[...], preferred_element_type=_f32) + ab2[...]
        v = jnp.maximum(jnp.dot(g, cw0[...], preferred_element_type=_f32)
                        + cb0[...], 0.0)
        v = jnp.maximum(jnp.dot(v, cw1[...], preferred_element_type=_f32)
                        + cb1[...], 0.0)
        val_ref[...] = jnp.dot(v, cw2[...], preferred_element_type=_f32) + cb2[...]


def _tc_head(x4_st, aW0, ab0, aW1, ab1, aW2, ab2, cW0, cb0, cW1, cb1, cW2p, cb2b):
    full = lambda shape: pl.BlockSpec(shape, lambda i: (0, 0))
    return pl.pallas_call(
        _head_body,
        grid=(NRB,),
        in_specs=[
            pl.BlockSpec((RB, 128), lambda i: (i, 0)),
            pl.BlockSpec((RB, 128), lambda i: (NRB + i, 0)),
            full((256, 512)), full((1, 512)),
            full((512, 256)), full((1, 256)),
            full((256, 1024)), full((1, 1024)),
            full((256, 512)), full((1, 512)),
            full((512, 256)), full((1, 256)),
            full((256, 128)), full((1, 128)),
        ],
        out_specs=[full((1, 1024)), full((1, 128))],
        out_shape=[jax.ShapeDtypeStruct((1, 1024), _f32),
                   jax.ShapeDtypeStruct((1, 128), _f32)],
        scratch_shapes=[pltpu.VMEM((1, 256), _f32)],
    )(x4_st, x4_st, aW0, ab0.reshape(1, -1), aW1, ab1.reshape(1, -1),
      aW2, ab2.reshape(1, -1), cW0, cb0.reshape(1, -1), cW1, cb1.reshape(1, -1),
      cW2p, cb2b)


def _stack(x, ncn):
    n, d = x.shape
    fc = d // ncn
    return x.reshape(n, ncn, fc).transpose(1, 0, 2).reshape(ncn * n, fc)


def kernel(node_features, edge_index,
           gnn_W0, gnn_b0, gnn_W1, gnn_b1, gnn_W2, gnn_b2, gnn_W3, gnn_b3,
           actor_W0, actor_b0, actor_W1, actor_b1, actor_W2, actor_b2,
           critic_W0, critic_b0, critic_W1, critic_b1, critic_W2, critic_b2):
    src = edge_index[0]
    dst = edge_index[1]

    offs2 = (N * jnp.arange(2, dtype=jnp.int32))[:, None]
    offs4 = (N * jnp.arange(4, dtype=jnp.int32))[:, None]
    srcs2 = (src[None, :] + offs2).reshape(-1)
    srcs4 = (src[None, :] + offs4).reshape(-1)
    zeros128 = jnp.zeros((ROWS_IO, 128), _f32)
    ones128 = jnp.ones((EB, 128), _f32)
    cW2p = jnp.pad(critic_W2, ((0, 0), (0, 127)))
    cb2b = jnp.broadcast_to(critic_b2.reshape(1, 1), (1, 128))

    degp = _deg_kernel(dst, ones128, zeros128)

    s0_parts = _segsum_split(src, dst, node_features, zeros128)
    x1_st = _tc_layer0(node_features, s0_parts, degp, gnn_W0, gnn_b0, 2)

    s1_st = _segsum(srcs2, dst, x1_st, zeros128, 2, 128)
    x2_st = _tc_layer(x1_st, s1_st, degp, gnn_W1, gnn_b1, 2, 128, 4)

    s2_st = _segsum(srcs4, dst, x2_st, zeros128, 4, 128)
    x3_st = _tc_layer(x2_st, s2_st, degp, gnn_W2, gnn_b2, 4, 128, 4)

    s3_st = _segsum(srcs4, dst, x3_st, zeros128, 4, 128)
    x4_st = _tc_layer(x3_st, s3_st, degp, gnn_W3, gnn_b3, 4, 128, 2)

    logits, val = _tc_head(x4_st, actor_W0, actor_b0, actor_W1, actor_b1,
                           actor_W2, actor_b2, critic_W0, critic_b0,
                           critic_W1, critic_b1, cW2p, cb2b)
    return logits, val[:, 0]
